# baseline (speedup 1.0000x reference)
import functools

import jax
import jax.numpy as jnp
from jax import lax
from jax.experimental import pallas as pl
from jax.experimental.pallas import tpu as pltpu


def _xw_kernel(x_ref, w1_ref, xw_ref):
    xw_ref[...] = jnp.dot(x_ref[...], w1_ref[...],
                          preferred_element_type=jnp.float32
                          ).astype(jnp.bfloat16)


def _t_kernel(adj_ref, xw_ref, w2_ref, t_ref, *, num_steps, h1, h2):
    adjb = adj_ref[...].astype(jnp.bfloat16)
    h = jnp.maximum(jnp.dot(adjb, xw_ref[...],
                            preferred_element_type=jnp.float32),
                    0.0).astype(jnp.bfloat16)
    for s in range(num_steps):
        ts = jnp.dot(h[:, s * h1:(s + 1) * h1], w2_ref[s],
                     preferred_element_type=jnp.float32)
        t_ref[:, s * h2:(s + 1) * h2] = ts.astype(jnp.bfloat16)


def _a_kernel(adj_ref, t_ref, a_ref, cs_ref):
    adjb = adj_ref[...].astype(jnp.bfloat16)
    a = jnp.maximum(jnp.dot(adjb, t_ref[...],
                            preferred_element_type=jnp.float32), 0.0)
    a_ref[...] = a
    cs_ref[...] = jnp.sum(a, axis=0).reshape(1, 1, -1)


def _gram_kernel(scal_ref, a_rows_ref, a_full_ref, cs_ref, out_ref,
                 *, num_steps, h2, inv_n):
    mean = (jnp.sum(cs_ref[...], axis=(0, 1)) * inv_n)[None, :]
    ar = ((a_rows_ref[...] - mean) * 0.5).astype(jnp.bfloat16)
    af = (a_full_ref[...] - mean).astype(jnp.bfloat16)
    acc = None
    for s in range(num_steps):
        lr = ar[:, s * h2:(s + 1) * h2]
        lc = af[:, s * h2:(s + 1) * h2]
        logits = lax.dot_general(lr, lc, (((1,), (1,)), ((), ())),
                                 preferred_element_type=jnp.float32)
        term = scal_ref[s] * jnp.tanh(logits)
        acc = term if acc is None else acc + term
    out_ref[...] = acc + scal_ref[num_steps]


def kernel(X, adj, w1_stack, w2_stack, sqrt_one_minus_alphas_cumprod,
           cumulative_sqrt_one_minus_alphas_cumprod):
    time_step, timesteps = 1, 4
    N, F_in = X.shape
    H1 = w1_stack.shape[-1]
    H2 = w2_stack.shape[-1]
    S = timesteps + 1 - time_step
    SH1, SH2 = S * H1, S * H2
    cdt = jnp.bfloat16

    denom = cumulative_sqrt_one_minus_alphas_cumprod[time_step - 1].astype(
        jnp.float32)
    coefs = (sqrt_one_minus_alphas_cumprod[time_step - 1: timesteps]
             .astype(jnp.float32) / denom)
    half_coefs = 0.5 * coefs
    scalars = jnp.concatenate([half_coefs, jnp.sum(half_coefs)[None]])

    Xb = X.astype(cdt)
    w1s = w1_stack[time_step: timesteps + 1].astype(cdt)
    w1_cat = jnp.transpose(w1s, (1, 0, 2)).reshape(F_in, SH1)
    w2s = w2_stack[time_step: timesteps + 1].astype(cdt)

    TM = 256 if N % 256 == 0 else 128
    G = N // TM
    par = ("parallel",)

    xw = pl.pallas_call(
        _xw_kernel,
        out_shape=jax.ShapeDtypeStruct((N, SH1), cdt),
        grid=(2,),
        in_specs=[pl.BlockSpec((N // 2, F_in), lambda i: (i, 0)),
                  pl.BlockSpec((F_in, SH1), lambda i: (0, 0))],
        out_specs=pl.BlockSpec((N // 2, SH1), lambda i: (i, 0)),
        compiler_params=pltpu.CompilerParams(
            dimension_semantics=par, vmem_limit_bytes=24 << 20),
    )(Xb, w1_cat)

    t = pl.pallas_call(
        functools.partial(_t_kernel, num_steps=S, h1=H1, h2=H2),
        out_shape=jax.ShapeDtypeStruct((N, SH2), cdt),
        grid=(G,),
        in_specs=[pl.BlockSpec((TM, N), lambda i: (i, 0)),
                  pl.BlockSpec((N, SH1), lambda i: (0, 0)),
                  pl.BlockSpec((S, H1, H2), lambda i: (0, 0, 0))],
        out_specs=pl.BlockSpec((TM, SH2), lambda i: (i, 0)),
        compiler_params=pltpu.CompilerParams(
            dimension_semantics=par, vmem_limit_bytes=40 << 20),
    )(adj, xw, w2s)

    a, cs = pl.pallas_call(
        _a_kernel,
        out_shape=(jax.ShapeDtypeStruct((N, SH2), jnp.float32),
                   jax.ShapeDtypeStruct((G, 1, SH2), jnp.float32)),
        grid=(G,),
        in_specs=[pl.BlockSpec((TM, N), lambda i: (i, 0)),
                  pl.BlockSpec((N, SH2), lambda i: (0, 0))],
        out_specs=(pl.BlockSpec((TM, SH2), lambda i: (i, 0)),
                   pl.BlockSpec((1, 1, SH2), lambda i: (i, 0, 0))),
        compiler_params=pltpu.CompilerParams(
            dimension_semantics=par, vmem_limit_bytes=40 << 20),
    )(adj, t)

    out = pl.pallas_call(
        functools.partial(_gram_kernel, num_steps=S, h2=H2, inv_n=1.0 / N),
        out_shape=jax.ShapeDtypeStruct((N, N), jnp.float32),
        grid_spec=pltpu.PrefetchScalarGridSpec(
            num_scalar_prefetch=1,
            grid=(G,),
            in_specs=[pl.BlockSpec((TM, SH2), lambda i, scal: (i, 0)),
                      pl.BlockSpec((N, SH2), lambda i, scal: (0, 0)),
                      pl.BlockSpec((G, 1, SH2), lambda i, scal: (0, 0, 0))],
            out_specs=pl.BlockSpec((TM, N), lambda i, scal: (i, 0)),
        ),
        compiler_params=pltpu.CompilerParams(
            dimension_semantics=par, vmem_limit_bytes=56 << 20),
    )(scalars, a, a, cs)

    return out


# speedup vs baseline: 1.0324x; 1.0324x over previous
import functools

import jax
import jax.numpy as jnp
from jax import lax
from jax.experimental import pallas as pl
from jax.experimental.pallas import tpu as pltpu


def _feat_kernel(adj_ref, x_ref, w1_ref, w2_ref, acc_ref, xw_scr,
                 *, num_steps, h1, h2, jsteps):
    j = pl.program_id(1)

    @pl.when(j == 0)
    def _():
        xw_scr[...] = jnp.dot(x_ref[...], w1_ref[...],
                              preferred_element_type=jnp.float32
                              ).astype(jnp.bfloat16)

    adjb = adj_ref[...].astype(jnp.bfloat16)
    h = jnp.maximum(jnp.dot(adjb, xw_scr[...],
                            preferred_element_type=jnp.float32),
                    0.0).astype(jnp.bfloat16)
    ts = [jnp.dot(h[:, s * h1:(s + 1) * h1], w2_ref[s],
                  preferred_element_type=jnp.float32).astype(jnp.bfloat16)
          for s in range(num_steps)]
    t_tile = jnp.concatenate(ts, axis=1)

    part = lax.dot_general(adjb, t_tile, (((0,), (0,)), ((), ())),
                           preferred_element_type=jnp.float32)

    @pl.when(j == 0)
    def _():
        acc_ref[0] = part

    @pl.when(j > 0)
    def _():
        acc_ref[0] += part


def _gram_kernel(scal_ref, acc_ref, out_ref, af_scr,
                 *, num_steps, h2, inv_n, tm, jsteps):
    j = pl.program_id(1)

    @pl.when(j == 0)
    def _():
        a = jnp.maximum(acc_ref[0] + acc_ref[1], 0.0)
        a = a - jnp.sum(a, axis=0, keepdims=True) * inv_n
        af_scr[...] = a.astype(jnp.bfloat16)

    tile = pl.program_id(0) * jsteps + j
    rows = af_scr[pl.ds(tile * tm, tm), :]
    ar = rows * jnp.bfloat16(0.5)
    acc = None
    for s in range(num_steps):
        lr = ar[:, s * h2:(s + 1) * h2]
        lc = af_scr[:, s * h2:(s + 1) * h2]
        logits = lax.dot_general(lr, lc, (((1,), (1,)), ((), ())),
                                 preferred_element_type=jnp.float32)
        term = scal_ref[s] * jnp.tanh(logits)
        acc = term if acc is None else acc + term
    out_ref[...] = acc + scal_ref[num_steps]


def kernel(X, adj, w1_stack, w2_stack, sqrt_one_minus_alphas_cumprod,
           cumulative_sqrt_one_minus_alphas_cumprod):
    time_step, timesteps = 1, 4
    N, F_in = X.shape
    H1 = w1_stack.shape[-1]
    H2 = w2_stack.shape[-1]
    S = timesteps + 1 - time_step
    SH1, SH2 = S * H1, S * H2
    cdt = jnp.bfloat16

    denom = cumulative_sqrt_one_minus_alphas_cumprod[time_step - 1].astype(
        jnp.float32)
    coefs = (sqrt_one_minus_alphas_cumprod[time_step - 1: timesteps]
             .astype(jnp.float32) / denom)
    half_coefs = 0.5 * coefs
    scalars = jnp.concatenate([half_coefs, jnp.sum(half_coefs)[None]])

    Xb = X.astype(cdt)
    w1s = w1_stack[time_step: timesteps + 1].astype(cdt)
    w1_cat = jnp.transpose(w1s, (1, 0, 2)).reshape(F_in, SH1)
    w2s = w2_stack[time_step: timesteps + 1].astype(cdt)

    TM = 256 if N % 512 == 0 else 128
    G = N // TM
    J = G // 2
    sem = ("parallel", "arbitrary")

    acc = pl.pallas_call(
        functools.partial(_feat_kernel, num_steps=S, h1=H1, h2=H2, jsteps=J),
        out_shape=jax.ShapeDtypeStruct((2, N, SH2), jnp.float32),
        grid=(2, J),
        in_specs=[pl.BlockSpec((TM, N), lambda i, j: (i * J + j, 0)),
                  pl.BlockSpec((N, F_in), lambda i, j: (0, 0)),
                  pl.BlockSpec((F_in, SH1), lambda i, j: (0, 0)),
                  pl.BlockSpec((S, H1, H2), lambda i, j: (0, 0, 0))],
        out_specs=pl.BlockSpec((1, N, SH2), lambda i, j: (i, 0, 0)),
        scratch_shapes=[pltpu.VMEM((N, SH1), cdt)],
        compiler_params=pltpu.CompilerParams(
            dimension_semantics=sem, vmem_limit_bytes=48 << 20),
    )(adj, Xb, w1_cat, w2s)

    out = pl.pallas_call(
        functools.partial(_gram_kernel, num_steps=S, h2=H2, inv_n=1.0 / N,
                          tm=TM, jsteps=J),
        out_shape=jax.ShapeDtypeStruct((N, N), jnp.float32),
        grid_spec=pltpu.PrefetchScalarGridSpec(
            num_scalar_prefetch=1,
            grid=(2, J),
            in_specs=[pl.BlockSpec((2, N, SH2), lambda i, j, scal: (0, 0, 0))],
            out_specs=pl.BlockSpec((TM, N), lambda i, j, scal: (i * J + j, 0)),
            scratch_shapes=[pltpu.VMEM((N, SH2), cdt)],
        ),
        compiler_params=pltpu.CompilerParams(
            dimension_semantics=sem, vmem_limit_bytes=56 << 20),
    )(scalars, acc)

    return out


# speedup vs baseline: 1.1458x; 1.1098x over previous
import functools

import jax
import jax.numpy as jnp
from jax import lax
from jax.experimental import pallas as pl
from jax.experimental.pallas import tpu as pltpu


def _fused_kernel(scal_ref, adj_ref, x_ref, w1_ref, w2_ref, out_ref,
                  xw_scr, aacc_scr, af_scr, *, num_steps, h1, h2, inv_n,
                  tm, gtiles):
    i = pl.program_id(0)

    @pl.when(i == 0)
    def _():
        xw_scr[...] = jnp.dot(x_ref[...], w1_ref[...],
                              preferred_element_type=jnp.float32
                              ).astype(jnp.bfloat16)

    @pl.when(i < gtiles)
    def _():
        adjb = adj_ref[...].astype(jnp.bfloat16)
        h = jnp.maximum(jnp.dot(adjb, xw_scr[...],
                                preferred_element_type=jnp.float32),
                        0.0).astype(jnp.bfloat16)
        ts = [jnp.dot(h[:, s * h1:(s + 1) * h1], w2_ref[s],
                      preferred_element_type=jnp.float32).astype(jnp.bfloat16)
              for s in range(num_steps)]
        t_tile = jnp.concatenate(ts, axis=1)

        part = lax.dot_general(adjb, t_tile, (((0,), (0,)), ((), ())),
                               preferred_element_type=jnp.float32)

        @pl.when(i == 0)
        def _():
            aacc_scr[...] = part

        @pl.when(i > 0)
        def _():
            aacc_scr[...] += part

    @pl.when(i == gtiles)
    def _():
        a = jnp.maximum(aacc_scr[...], 0.0)
        a = a - jnp.sum(a, axis=0, keepdims=True) * inv_n
        af_scr[...] = a.astype(jnp.bfloat16)

    @pl.when(i >= gtiles)
    def _():
        tile = i - gtiles
        rows = af_scr[pl.ds(tile * tm, tm), :]
        ar = rows * jnp.bfloat16(0.5)
        acc = None
        for s in range(num_steps):
            lr = ar[:, s * h2:(s + 1) * h2]
            lc = af_scr[:, s * h2:(s + 1) * h2]
            logits = lax.dot_general(lr, lc, (((1,), (1,)), ((), ())),
                                     preferred_element_type=jnp.float32)
            term = scal_ref[s] * jnp.tanh(logits)
            acc = term if acc is None else acc + term
        out_ref[...] = acc + scal_ref[num_steps]


def kernel(X, adj, w1_stack, w2_stack, sqrt_one_minus_alphas_cumprod,
           cumulative_sqrt_one_minus_alphas_cumprod):
    time_step, timesteps = 1, 4
    N, F_in = X.shape
    H1 = w1_stack.shape[-1]
    H2 = w2_stack.shape[-1]
    S = timesteps + 1 - time_step
    SH1, SH2 = S * H1, S * H2
    cdt = jnp.bfloat16

    denom = cumulative_sqrt_one_minus_alphas_cumprod[time_step - 1].astype(
        jnp.float32)
    coefs = (sqrt_one_minus_alphas_cumprod[time_step - 1: timesteps]
             .astype(jnp.float32) / denom)
    half_coefs = 0.5 * coefs
    scalars = jnp.concatenate([half_coefs, jnp.sum(half_coefs)[None]])

    Xb = X.astype(cdt)
    w1s = w1_stack[time_step: timesteps + 1].astype(cdt)
    w1_cat = jnp.transpose(w1s, (1, 0, 2)).reshape(F_in, SH1)
    w2s = w2_stack[time_step: timesteps + 1].astype(cdt)

    TM = 256 if N % 256 == 0 else 128
    G = N // TM

    out = pl.pallas_call(
        functools.partial(_fused_kernel, num_steps=S, h1=H1, h2=H2,
                          inv_n=1.0 / N, tm=TM, gtiles=G),
        out_shape=jax.ShapeDtypeStruct((N, N), jnp.float32),
        grid_spec=pltpu.PrefetchScalarGridSpec(
            num_scalar_prefetch=1,
            grid=(2 * G,),
            in_specs=[
                pl.BlockSpec((TM, N),
                             lambda i, scal: (jnp.minimum(i, G - 1), 0)),
                pl.BlockSpec((N, F_in), lambda i, scal: (0, 0)),
                pl.BlockSpec((F_in, SH1), lambda i, scal: (0, 0)),
                pl.BlockSpec((S, H1, H2), lambda i, scal: (0, 0, 0)),
            ],
            out_specs=pl.BlockSpec(
                (TM, N), lambda i, scal: (jnp.maximum(i - G, 0), 0)),
            scratch_shapes=[pltpu.VMEM((N, SH1), cdt),
                            pltpu.VMEM((N, SH2), jnp.float32),
                            pltpu.VMEM((N, SH2), cdt)],
        ),
        compiler_params=pltpu.CompilerParams(
            dimension_semantics=("arbitrary",), vmem_limit_bytes=56 << 20),
    )(scalars, adj, Xb, w1_cat, w2s)

    return out


# speedup vs baseline: 1.2276x; 1.0714x over previous
import functools

import jax
import jax.numpy as jnp
from jax import lax
from jax.experimental import pallas as pl
from jax.experimental.pallas import tpu as pltpu


def _fused_kernel(scal_ref, adj_ref, x_ref, w1_ref, w2_ref, out_ref,
                  xw_scr, adj_scr, t_scr, af_scr, *, num_steps, h1, h2,
                  inv_n, tm, gtiles):
    i = pl.program_id(0)

    @pl.when(i == 0)
    def _():
        xw_scr[...] = jnp.dot(x_ref[...], w1_ref[...],
                              preferred_element_type=jnp.float32
                              ).astype(jnp.bfloat16)

    @pl.when(i < gtiles)
    def _():
        adjb = adj_ref[...].astype(jnp.bfloat16)
        adj_scr[pl.ds(i * tm, tm), :] = adjb
        h = jnp.maximum(jnp.dot(adjb, xw_scr[...],
                                preferred_element_type=jnp.float32),
                        0.0).astype(jnp.bfloat16)
        for s in range(num_steps):
            ts = jnp.dot(h[:, s * h1:(s + 1) * h1], w2_ref[s],
                         preferred_element_type=jnp.float32)
            t_scr[pl.ds(i * tm, tm), s * h2:(s + 1) * h2] = (
                ts.astype(jnp.bfloat16))

    @pl.when(i == gtiles)
    def _():
        a = jnp.maximum(jnp.dot(adj_scr[...], t_scr[...],
                                preferred_element_type=jnp.float32), 0.0)
        a = a - jnp.sum(a, axis=0, keepdims=True) * inv_n
        af_scr[...] = a.astype(jnp.bfloat16)

    @pl.when(i >= gtiles)
    def _():
        tile = i - gtiles
        rows = af_scr[pl.ds(tile * tm, tm), :]
        ar = rows * jnp.bfloat16(0.5)
        acc = None
        for s in range(num_steps):
            lr = ar[:, s * h2:(s + 1) * h2]
            lc = af_scr[:, s * h2:(s + 1) * h2]
            logits = lax.dot_general(lr, lc, (((1,), (1,)), ((), ())),
                                     preferred_element_type=jnp.float32)
            term = scal_ref[s] * jnp.tanh(logits)
            acc = term if acc is None else acc + term
        out_ref[...] = acc + scal_ref[num_steps]


def kernel(X, adj, w1_stack, w2_stack, sqrt_one_minus_alphas_cumprod,
           cumulative_sqrt_one_minus_alphas_cumprod):
    time_step, timesteps = 1, 4
    N, F_in = X.shape
    H1 = w1_stack.shape[-1]
    H2 = w2_stack.shape[-1]
    S = timesteps + 1 - time_step
    SH1, SH2 = S * H1, S * H2
    cdt = jnp.bfloat16

    denom = cumulative_sqrt_one_minus_alphas_cumprod[time_step - 1].astype(
        jnp.float32)
    coefs = (sqrt_one_minus_alphas_cumprod[time_step - 1: timesteps]
             .astype(jnp.float32) / denom)
    half_coefs = 0.5 * coefs
    scalars = jnp.concatenate([half_coefs, jnp.sum(half_coefs)[None]])

    Xb = X.astype(cdt)
    w1s = w1_stack[time_step: timesteps + 1].astype(cdt)
    w1_cat = jnp.transpose(w1s, (1, 0, 2)).reshape(F_in, SH1)
    w2s = w2_stack[time_step: timesteps + 1].astype(cdt)

    TM = 256 if N % 256 == 0 else 128
    G = N // TM

    out = pl.pallas_call(
        functools.partial(_fused_kernel, num_steps=S, h1=H1, h2=H2,
                          inv_n=1.0 / N, tm=TM, gtiles=G),
        out_shape=jax.ShapeDtypeStruct((N, N), jnp.float32),
        grid_spec=pltpu.PrefetchScalarGridSpec(
            num_scalar_prefetch=1,
            grid=(2 * G,),
            in_specs=[
                pl.BlockSpec((TM, N),
                             lambda i, scal: (jnp.minimum(i, G - 1), 0)),
                pl.BlockSpec((N, F_in), lambda i, scal: (0, 0)),
                pl.BlockSpec((F_in, SH1), lambda i, scal: (0, 0)),
                pl.BlockSpec((S, H1, H2), lambda i, scal: (0, 0, 0)),
            ],
            out_specs=pl.BlockSpec(
                (TM, N), lambda i, scal: (jnp.maximum(i - G, 0), 0)),
            scratch_shapes=[pltpu.VMEM((N, SH1), cdt),
                            pltpu.VMEM((N, N), cdt),
                            pltpu.VMEM((N, SH2), cdt),
                            pltpu.VMEM((N, SH2), cdt)],
        ),
        compiler_params=pltpu.CompilerParams(
            dimension_semantics=("arbitrary",), vmem_limit_bytes=56 << 20),
    )(scalars, adj, Xb, w1_cat, w2s)

    return out


# speedup vs baseline: 1.2820x; 1.0443x over previous
import functools

import jax
import jax.numpy as jnp
from jax import lax
from jax.experimental import pallas as pl
from jax.experimental.pallas import tpu as pltpu


def _fused_kernel(scal_ref, adj_ref, x_ref, w1_ref, w2_ref, out_ref,
                  xw_scr, adj_scr, t_scr, af_scr, *, num_steps, h1, h2,
                  inv_n, tma, tmb, ga, gtiles):
    i = pl.program_id(0)

    @pl.when(i == 0)
    def _():
        xw_scr[...] = jnp.dot(x_ref[...], w1_ref[...],
                              preferred_element_type=jnp.float32
                              ).astype(jnp.bfloat16)

    @pl.when(i < ga)
    def _():
        adjb = adj_ref[...].astype(jnp.bfloat16)
        adj_scr[pl.ds(i * tma, tma), :] = adjb
        h = jnp.maximum(jnp.dot(adjb, xw_scr[...],
                                preferred_element_type=jnp.float32),
                        0.0).astype(jnp.bfloat16)
        for s in range(num_steps):
            ts = jnp.dot(h[:, s * h1:(s + 1) * h1], w2_ref[s],
                         preferred_element_type=jnp.float32)
            t_scr[pl.ds(i * tma, tma), s * h2:(s + 1) * h2] = (
                ts.astype(jnp.bfloat16))

    @pl.when(i == ga)
    def _():
        a = jnp.maximum(jnp.dot(adj_scr[...], t_scr[...],
                                preferred_element_type=jnp.float32), 0.0)
        a = a - jnp.sum(a, axis=0, keepdims=True) * inv_n
        af_scr[...] = a.astype(jnp.bfloat16)

    @pl.when(i >= ga)
    def _():
        tile = i - ga
        rows = af_scr[pl.ds(tile * tmb, tmb), :]
        ar = rows * jnp.bfloat16(0.5)
        acc = None
        for s in range(num_steps):
            lr = ar[:, s * h2:(s + 1) * h2]
            lc = af_scr[:, s * h2:(s + 1) * h2]
            logits = lax.dot_general(lr, lc, (((1,), (1,)), ((), ())),
                                     preferred_element_type=jnp.float32)
            term = scal_ref[s] * jnp.tanh(logits)
            acc = term if acc is None else acc + term
        out_ref[...] = acc + scal_ref[num_steps]


def kernel(X, adj, w1_stack, w2_stack, sqrt_one_minus_alphas_cumprod,
           cumulative_sqrt_one_minus_alphas_cumprod):
    time_step, timesteps = 1, 4
    N, F_in = X.shape
    H1 = w1_stack.shape[-1]
    H2 = w2_stack.shape[-1]
    S = timesteps + 1 - time_step
    SH1, SH2 = S * H1, S * H2
    cdt = jnp.bfloat16

    denom = cumulative_sqrt_one_minus_alphas_cumprod[time_step - 1].astype(
        jnp.float32)
    coefs = (sqrt_one_minus_alphas_cumprod[time_step - 1: timesteps]
             .astype(jnp.float32) / denom)
    half_coefs = 0.5 * coefs
    scalars = jnp.concatenate([half_coefs, jnp.sum(half_coefs)[None]])

    Xb = X.astype(cdt)
    w1s = w1_stack[time_step: timesteps + 1].astype(cdt)
    w1_cat = jnp.transpose(w1s, (1, 0, 2)).reshape(F_in, SH1)
    w2s = w2_stack[time_step: timesteps + 1].astype(cdt)

    TMA = 512 if N % 512 == 0 else 256
    TMB = 256 if N % 256 == 0 else 128
    GA = N // TMA
    GB = N // TMB

    out = pl.pallas_call(
        functools.partial(_fused_kernel, num_steps=S, h1=H1, h2=H2,
                          inv_n=1.0 / N, tma=TMA, tmb=TMB, ga=GA, gtiles=GB),
        out_shape=jax.ShapeDtypeStruct((N, N), jnp.float32),
        grid_spec=pltpu.PrefetchScalarGridSpec(
            num_scalar_prefetch=1,
            grid=(GA + GB,),
            in_specs=[
                pl.BlockSpec((TMA, N),
                             lambda i, scal: (jnp.minimum(i, GA - 1), 0)),
                pl.BlockSpec((N, F_in), lambda i, scal: (0, 0)),
                pl.BlockSpec((F_in, SH1), lambda i, scal: (0, 0)),
                pl.BlockSpec((S, H1, H2), lambda i, scal: (0, 0, 0)),
            ],
            out_specs=pl.BlockSpec(
                (TMB, N), lambda i, scal: (jnp.maximum(i - GA, 0), 0)),
            scratch_shapes=[pltpu.VMEM((N, SH1), cdt),
                            pltpu.VMEM((N, N), cdt),
                            pltpu.VMEM((N, SH2), cdt),
                            pltpu.VMEM((N, SH2), cdt)],
        ),
        compiler_params=pltpu.CompilerParams(
            dimension_semantics=("arbitrary",), vmem_limit_bytes=56 << 20),
    )(scalars, adj, Xb, w1_cat, w2s)

    return out


# speedup vs baseline: 1.2951x; 1.0102x over previous
import functools

import jax
import jax.numpy as jnp
from jax import lax
from jax.experimental import pallas as pl
from jax.experimental.pallas import tpu as pltpu


def _fused_kernel(scal_ref, adj_ref, x_ref, w1_ref, w2_ref, out_ref,
                  xw_scr, adj_scr, t_scr, af_scr, *, num_steps, h1, h2,
                  inv_n, tma, tmb, ga, gtiles):
    i = pl.program_id(0)

    @pl.when(i == 0)
    def _():
        xw_scr[...] = jnp.dot(x_ref[...], w1_ref[...],
                              preferred_element_type=jnp.float32
                              ).astype(jnp.bfloat16)

    @pl.when(i < ga)
    def _():
        adjb = adj_ref[...].astype(jnp.bfloat16)
        adj_scr[pl.ds(i * tma, tma), :] = adjb
        h = jnp.maximum(jnp.dot(adjb, xw_scr[...],
                                preferred_element_type=jnp.float32),
                        0.0).astype(jnp.bfloat16)
        for s in range(num_steps):
            ts = jnp.dot(h[:, s * h1:(s + 1) * h1], w2_ref[s],
                         preferred_element_type=jnp.float32)
            t_scr[pl.ds(i * tma, tma), s * h2:(s + 1) * h2] = (
                ts.astype(jnp.bfloat16))

    @pl.when(i == ga)
    def _():
        a = jnp.maximum(jnp.dot(adj_scr[...], t_scr[...],
                                preferred_element_type=jnp.float32), 0.0)
        a = a - jnp.sum(a, axis=0, keepdims=True) * inv_n
        af_scr[...] = a.astype(jnp.bfloat16)

    @pl.when(i >= ga)
    def _():
        tile = i - ga
        rows = af_scr[pl.ds(tile * tmb, tmb), :]
        ar = rows * jnp.bfloat16(0.5)
        acc = None
        for s in range(num_steps):
            lr = ar[:, s * h2:(s + 1) * h2]
            lc = af_scr[:, s * h2:(s + 1) * h2]
            logits = lax.dot_general(lr, lc, (((1,), (1,)), ((), ())),
                                     preferred_element_type=jnp.float32)
            term = scal_ref[s] * jnp.tanh(logits)
            acc = term if acc is None else acc + term
        out_ref[...] = acc + scal_ref[num_steps]


def kernel(X, adj, w1_stack, w2_stack, sqrt_one_minus_alphas_cumprod,
           cumulative_sqrt_one_minus_alphas_cumprod):
    time_step, timesteps = 1, 4
    N, F_in = X.shape
    H1 = w1_stack.shape[-1]
    H2 = w2_stack.shape[-1]
    S = timesteps + 1 - time_step
    SH1, SH2 = S * H1, S * H2
    cdt = jnp.bfloat16

    denom = cumulative_sqrt_one_minus_alphas_cumprod[time_step - 1].astype(
        jnp.float32)
    coefs = (sqrt_one_minus_alphas_cumprod[time_step - 1: timesteps]
             .astype(jnp.float32) / denom)
    half_coefs = 0.5 * coefs
    scalars = jnp.concatenate([half_coefs, jnp.sum(half_coefs)[None]])

    Xb = X.astype(cdt)
    w1s = w1_stack[time_step: timesteps + 1].astype(cdt)
    w1_cat = jnp.transpose(w1s, (1, 0, 2)).reshape(F_in, SH1)
    w2s = w2_stack[time_step: timesteps + 1].astype(cdt)

    TMA = 512 if N % 512 == 0 else 256
    TMB = 512 if N % 512 == 0 else 256
    GA = N // TMA
    GB = N // TMB

    out = pl.pallas_call(
        functools.partial(_fused_kernel, num_steps=S, h1=H1, h2=H2,
                          inv_n=1.0 / N, tma=TMA, tmb=TMB, ga=GA, gtiles=GB),
        out_shape=jax.ShapeDtypeStruct((N, N), jnp.float32),
        grid_spec=pltpu.PrefetchScalarGridSpec(
            num_scalar_prefetch=1,
            grid=(GA + GB,),
            in_specs=[
                pl.BlockSpec((TMA, N),
                             lambda i, scal: (jnp.minimum(i, GA - 1), 0)),
                pl.BlockSpec((N, F_in), lambda i, scal: (0, 0)),
                pl.BlockSpec((F_in, SH1), lambda i, scal: (0, 0)),
                pl.BlockSpec((S, H1, H2), lambda i, scal: (0, 0, 0)),
            ],
            out_specs=pl.BlockSpec(
                (TMB, N), lambda i, scal: (jnp.maximum(i - GA, 0), 0)),
            scratch_shapes=[pltpu.VMEM((N, SH1), cdt),
                            pltpu.VMEM((N, N), cdt),
                            pltpu.VMEM((N, SH2), cdt),
                            pltpu.VMEM((N, SH2), cdt)],
        ),
        compiler_params=pltpu.CompilerParams(
            dimension_semantics=("arbitrary",), vmem_limit_bytes=60000 * 1024),
    )(scalars, adj, Xb, w1_cat, w2s)

    return out
